# baseline (speedup 1.0000x reference)
import jax
import jax.numpy as jnp
from jax import lax
from jax.experimental import pallas as pl
from jax.experimental.pallas import tpu as pltpu
from jax.experimental.pallas import tpu_sc as plsc

_BATCH = 4096
_HIST = 200
_D = 64
_DP = 128
_NC = 2
_NS = 16
_NW = _NC * _NS
_BW = _BATCH // _NW
_NBUF = 5
_G = _HIST


def _body(xt_hbm, table_hbm, out_hbm, idx_v, rows, gsems, wsems):
    wid = lax.axis_index("s") * _NC + lax.axis_index("c")
    b0 = wid * _BW
    pltpu.sync_copy(xt_hbm.at[:, pl.ds(b0, _BW)], idx_v)

    def _gather_start(g, b):
        pltpu.async_copy(table_hbm.at[idx_v.at[g]], rows[b], gsems[b])

    for b in range(_NBUF):
        _gather_start(b, b)

    @pl.loop(0, _G, step=_NBUF)
    def _outer(g0):
        for b in range(_NBUF):
            g = g0 + b
            pltpu.make_async_copy(
                table_hbm.at[idx_v.at[g]], rows[b], gsems[b]
            ).wait()
            pltpu.async_copy(
                rows[b], out_hbm.at[pl.ds(b0, _BW), g, :], wsems[b]
            )
            @pl.when(g + _NBUF < _G)
            def _():
                pltpu.make_async_copy(
                    rows[b], out_hbm.at[pl.ds(b0, _BW), g, :], wsems[b]
                ).wait()
                _gather_start(g + _NBUF, b)

    for b in range(_NBUF):
        g_last = _G - _NBUF + b
        pltpu.make_async_copy(
            rows[b], out_hbm.at[pl.ds(b0, _BW), g_last, :], wsems[b]
        ).wait()


@jax.jit
def _lookup(xt, table_pad):
    mesh = plsc.VectorSubcoreMesh(core_axis_name="c", subcore_axis_name="s")
    return pl.kernel(
        _body,
        out_type=jax.ShapeDtypeStruct((_BATCH, _HIST, _DP), jnp.float32),
        mesh=mesh,
        scratch_types=[
            pltpu.VMEM((_HIST, _BW), jnp.int32),
            [pltpu.VMEM((_BW, _DP), jnp.float32) for _ in range(_NBUF)],
            [pltpu.SemaphoreType.DMA for _ in range(_NBUF)],
            [pltpu.SemaphoreType.DMA for _ in range(_NBUF)],
        ],
        compiler_params=pltpu.CompilerParams(use_tc_tiling_on_sc=True),
    )(xt, table_pad)


def kernel(x, table):
    table_pad = jnp.pad(table.T, ((0, _DP - _D), (0, 0))).T
    out = _lookup(x.T, table_pad)
    return out[:, :, :_D]
